# baseline (speedup 1.0000x reference)
import functools
import math

import jax
import jax.numpy as jnp
from jax import lax
from jax.experimental import pallas as pl
from jax.experimental.pallas import tpu as pltpu
from jax.experimental.pallas import tpu_sc as plsc

D_MODEL = 64
_SCALE = math.sqrt(D_MODEL)
_SG = 112


@functools.lru_cache(maxsize=None)
def _make_sc_gather(n_batch: int, n_seq: int, n_vocab: int):
    info = plsc.get_sparse_core_info()
    num_cores, num_subcores = info.num_cores, info.num_subcores
    num_workers = num_cores * num_subcores
    assert n_batch % num_workers == 0
    b_per_w = n_batch // num_workers
    stage = 32
    half = n_seq // 2
    mesh = plsc.VectorSubcoreMesh(core_axis_name="c", subcore_axis_name="s")

    @functools.partial(
        pl.kernel,
        mesh=mesh,
        out_type=jax.ShapeDtypeStruct((n_batch, n_seq, D_MODEL), jnp.float32),
        scratch_types=[
            pltpu.VMEM((stage, n_seq), jnp.int32),
            pltpu.VMEM((2, _SG), jnp.int32),
            pltpu.VMEM((2 * _SG, 2 * D_MODEL), jnp.float32),
            pltpu.VMEM((n_seq, D_MODEL), jnp.float32),
            pltpu.SemaphoreType.DMA,
            pltpu.SemaphoreType.DMA,
        ],
    )
    def sc_kernel(x_hbm, tbl2_hbm, out_hbm, idxs, idxp, gbuf, obuf, gsem, osem):
        wid = lax.axis_index("s") * num_cores + lax.axis_index("c")
        b0 = wid * b_per_w
        lanes = lax.iota(jnp.int32, 16)

        def do_stage(k, carry):
            pltpu.sync_copy(x_hbm.at[pl.ds(b0 + k * stage, stage)], idxs)

            def do_b(bl, c2):
                for h in range(2):
                    base = h * (n_seq - _SG)
                    for g in range(_SG // 16):
                        v = idxs[bl, pl.ds(base + g * 16, 16)]
                        idxp[h, pl.ds(g * 16, 16)] = v >> 1

                pltpu.async_copy(
                    tbl2_hbm.at[idxp.at[0]], gbuf.at[pl.ds(0, _SG)], gsem
                )
                pltpu.async_copy(
                    tbl2_hbm.at[idxp.at[1]], gbuf.at[pl.ds(_SG, _SG)], gsem
                )
                pltpu.make_async_copy(
                    tbl2_hbm.at[idxp.at[0]], gbuf.at[pl.ds(0, _SG)], gsem
                ).wait()
                pltpu.make_async_copy(
                    tbl2_hbm.at[idxp.at[1]], gbuf.at[pl.ds(_SG, _SG)], gsem
                ).wait()

                def sel_one(r, grow, par):
                    for g in range(D_MODEL // 16):
                        ve = gbuf[grow, pl.ds(g * 16, 16)]
                        vo = gbuf[grow, pl.ds(D_MODEL + g * 16, 16)]
                        obuf[r, pl.ds(g * 16, 16)] = (
                            jnp.where(par > 0, vo, ve) * _SCALE
                        )

                def sel_grp(j, c3):
                    vpar = jnp.bitwise_and(idxs[bl, pl.ds(j * 16, 16)], 1)
                    for l in range(16):
                        r = j * 16 + l
                        grow = jnp.where(r < _SG, r, r + (2 * _SG - n_seq))
                        sel_one(r, grow, vpar[l])
                    return c3

                lax.fori_loop(0, (n_seq - 8) // 16, sel_grp, 0)
                vpar_t = jnp.bitwise_and(idxs[bl, pl.ds(n_seq - 16, 16)], 1)
                for l in range(8, 16):
                    r = n_seq - 16 + l
                    sel_one(r, r + (2 * _SG - n_seq), vpar_t[l])

                b = b0 + k * stage + bl
                pltpu.make_async_copy(obuf, out_hbm.at[0], osem).wait()
                pltpu.async_copy(obuf, out_hbm.at[b], osem)
                return c2

            lax.fori_loop(0, stage, do_b, 0)
            return carry

        pltpu.async_copy(obuf, out_hbm.at[b0], osem)
        lax.fori_loop(0, b_per_w // stage, do_stage, 0)
        pltpu.make_async_copy(obuf, out_hbm.at[0], osem).wait()

    return sc_kernel


def kernel(x, table):
    n_batch, n_seq = x.shape
    n_vocab = table.shape[0]
    tbl2 = table.reshape(n_vocab // 2, 2 * D_MODEL)
    return _make_sc_gather(n_batch, n_seq, n_vocab)(x.astype(jnp.int32), tbl2)


# speedup vs baseline: 1.1971x; 1.1971x over previous
import functools
import math

import jax
import jax.numpy as jnp
from jax import lax
from jax.experimental import pallas as pl
from jax.experimental.pallas import tpu as pltpu
from jax.experimental.pallas import tpu_sc as plsc

D_MODEL = 64
_SCALE = math.sqrt(D_MODEL)
_BSTAGE = 32
_SG = 112

_SPLAT_DNUMS = lax.GatherDimensionNumbers(
    offset_dims=(), collapsed_slice_dims=(0,), start_index_map=(0,)
)


def _lane_splat(v, l):
    return lax.gather(
        v,
        jnp.full((16, 1), l, jnp.int32),
        _SPLAT_DNUMS,
        (1,),
        mode=lax.GatherScatterMode.PROMISE_IN_BOUNDS,
    )


@functools.lru_cache(maxsize=None)
def _make_sc_gather(n_batch: int, n_seq: int):
    info = plsc.get_sparse_core_info()
    num_cores, num_subcores = info.num_cores, info.num_subcores
    num_workers = num_cores * num_subcores
    assert n_batch % (num_workers * _BSTAGE) == 0
    b_per_w = n_batch // num_workers
    n_blocks = b_per_w // _BSTAGE
    stage_len = _BSTAGE * n_seq
    shift = 2 * _SG - n_seq
    mesh = plsc.VectorSubcoreMesh(core_axis_name="c", subcore_axis_name="s")

    @functools.partial(
        pl.kernel,
        mesh=mesh,
        out_type=jax.ShapeDtypeStruct((n_batch * n_seq, D_MODEL), jnp.float32),
        scratch_types=[
            pltpu.VMEM((stage_len,), jnp.int32),
            pltpu.VMEM((stage_len,), jnp.int32),
            pltpu.VMEM((2, _SG), jnp.int32),
            pltpu.VMEM((2, _SG), jnp.int32),
            pltpu.VMEM((2 * _SG, 2 * D_MODEL), jnp.float32),
            pltpu.VMEM((2 * _SG, 2 * D_MODEL), jnp.float32),
            pltpu.VMEM((n_seq, D_MODEL), jnp.float32),
            pltpu.VMEM((n_seq, D_MODEL), jnp.float32),
            pltpu.SemaphoreType.DMA,
            pltpu.SemaphoreType.DMA,
            pltpu.SemaphoreType.DMA,
            pltpu.SemaphoreType.DMA,
            pltpu.SemaphoreType.DMA,
        ],
    )
    def sc_kernel(idx_hbm, tbl2_hbm, out_hbm, idx0, idx1, idxp0, idxp1,
                  gbuf0, gbuf1, obuf0, obuf1, isem, gsem0, gsem1, osem0,
                  osem1):
        wid = lax.axis_index("s") * num_cores + lax.axis_index("c")
        b0 = wid * b_per_w
        j0 = b0 * n_seq
        idxb = (idx0, idx1)
        idxp = (idxp0, idxp1)
        gbuf = (gbuf0, gbuf1)
        obuf = (obuf0, obuf1)
        gsem = (gsem0, gsem1)
        osem = (osem0, osem1)

        def gather(k, bl, buf):
            src = idxb[k % 2]
            for h in range(2):
                base = bl * n_seq + h * (n_seq - _SG)
                for g in range(_SG // 16):
                    v = src[pl.ds(base + g * 16, 16)]
                    idxp[buf][h, pl.ds(g * 16, 16)] = v >> 1
            pltpu.async_copy(
                tbl2_hbm.at[idxp[buf].at[0]], gbuf[buf].at[pl.ds(0, _SG)],
                gsem[buf],
            )
            pltpu.async_copy(
                tbl2_hbm.at[idxp[buf].at[1]], gbuf[buf].at[pl.ds(_SG, _SG)],
                gsem[buf],
            )

        def wait_gather(buf):
            for h in range(2):
                pltpu.make_async_copy(
                    tbl2_hbm.at[idxp[buf].at[h]],
                    gbuf[buf].at[pl.ds(h * _SG, _SG)],
                    gsem[buf],
                ).wait()

        def sel_one(buf, r, grow, w_odd, w_even):
            for g in range(D_MODEL // 16):
                ve = gbuf[buf][grow, pl.ds(g * 16, 16)]
                vo = gbuf[buf][grow, pl.ds(D_MODEL + g * 16, 16)]
                obuf[buf][r, pl.ds(g * 16, 16)] = ve * w_even + vo * w_odd

        def sel(k, bl, buf):
            src = idxb[k % 2]

            def sel_grp(j, carry):
                vpar = jnp.bitwise_and(
                    src[pl.ds(bl * n_seq + j * 16, 16)], 1
                ).astype(jnp.float32) * _SCALE
                for l in range(16):
                    r = j * 16 + l
                    grow = jnp.where(r < _SG, r, r + shift)
                    w_odd = _lane_splat(vpar, l)
                    sel_one(buf, r, grow, w_odd, _SCALE - w_odd)
                return carry

            lax.fori_loop(0, (n_seq - 8) // 16, sel_grp, 0)
            vpar_t = jnp.bitwise_and(
                src[pl.ds(bl * n_seq + n_seq - 16, 16)], 1
            ).astype(jnp.float32) * _SCALE
            for l in range(8, 16):
                r = n_seq - 16 + l
                w_odd = _lane_splat(vpar_t, l)
                sel_one(buf, r, r + shift, w_odd, _SCALE - w_odd)

        pltpu.sync_copy(idx_hbm.at[pl.ds(j0, stage_len)], idx0)
        for k in range(n_blocks):
            if k + 1 < n_blocks:
                pltpu.async_copy(
                    idx_hbm.at[pl.ds(j0 + (k + 1) * stage_len, stage_len)],
                    idxb[(k + 1) % 2],
                    isem,
                )
            gather(k, 0, 0)

            def do_pair(p, carry):
                for buf in range(2):
                    bl = 2 * p + buf
                    wait_gather(buf)

                    @pl.when(bl + 1 < _BSTAGE)
                    def _():
                        gather(k, bl + 1, buf ^ 1)

                    if k > 0:
                        pltpu.make_async_copy(
                            obuf[buf], out_hbm.at[pl.ds(0, n_seq)], osem[buf]
                        ).wait()
                    else:
                        @pl.when(p >= 1)
                        def _():
                            pltpu.make_async_copy(
                                obuf[buf], out_hbm.at[pl.ds(0, n_seq)],
                                osem[buf],
                            ).wait()

                    sel(k, bl, buf)
                    b = b0 + k * _BSTAGE + bl
                    pltpu.async_copy(
                        obuf[buf], out_hbm.at[pl.ds(b * n_seq, n_seq)],
                        osem[buf],
                    )
                return carry

            lax.fori_loop(0, _BSTAGE // 2, do_pair, 0)
            if k + 1 < n_blocks:
                pltpu.make_async_copy(
                    idx_hbm.at[pl.ds(j0, stage_len)], idxb[(k + 1) % 2], isem
                ).wait()

        for buf in range(2):
            pltpu.make_async_copy(
                obuf[buf], out_hbm.at[pl.ds(0, n_seq)], osem[buf]
            ).wait()

    return sc_kernel


def kernel(x, table):
    n_batch, n_seq = x.shape
    flat_idx = x.reshape(n_batch * n_seq).astype(jnp.int32)
    tbl2 = table.reshape(table.shape[0] // 2, 2 * D_MODEL)
    out = _make_sc_gather(n_batch, n_seq)(flat_idx, tbl2)
    return out.reshape(n_batch, n_seq, D_MODEL)


# speedup vs baseline: 1.5157x; 1.2661x over previous
import functools
import math

import jax
import jax.numpy as jnp
from jax import lax
from jax.experimental import pallas as pl
from jax.experimental.pallas import tpu as pltpu
from jax.experimental.pallas import tpu_sc as plsc

D_MODEL = 64
_SCALE = math.sqrt(D_MODEL)
_BSTAGE = 32
_SG = 112

_SPLAT_DNUMS = lax.GatherDimensionNumbers(
    offset_dims=(), collapsed_slice_dims=(0,), start_index_map=(0,)
)


def _lane_splat(v, l):
    return lax.gather(
        v,
        jnp.full((16, 1), l, jnp.int32),
        _SPLAT_DNUMS,
        (1,),
        mode=lax.GatherScatterMode.PROMISE_IN_BOUNDS,
    )


@functools.lru_cache(maxsize=None)
def _make_sc_gather(n_batch: int, n_seq: int):
    info = plsc.get_sparse_core_info()
    num_cores, num_subcores = info.num_cores, info.num_subcores
    num_workers = num_cores * num_subcores
    assert n_batch % (num_workers * _BSTAGE) == 0
    b_per_w = n_batch // num_workers
    n_blocks = b_per_w // _BSTAGE
    stage_len = _BSTAGE * n_seq
    shift = 2 * _SG - n_seq
    mesh = plsc.VectorSubcoreMesh(core_axis_name="c", subcore_axis_name="s")

    @functools.partial(
        pl.kernel,
        mesh=mesh,
        out_type=jax.ShapeDtypeStruct((n_batch * n_seq, D_MODEL), jnp.float32),
        scratch_types=[
            pltpu.VMEM((stage_len,), jnp.int32),
            pltpu.VMEM((stage_len,), jnp.int32),
            pltpu.VMEM((2, _SG), jnp.int32),
            pltpu.VMEM((2, _SG), jnp.int32),
            pltpu.VMEM((2 * _SG, 2 * D_MODEL), jnp.float32),
            pltpu.VMEM((2 * _SG, 2 * D_MODEL), jnp.float32),
            pltpu.VMEM((n_seq, D_MODEL), jnp.float32),
            pltpu.VMEM((n_seq, D_MODEL), jnp.float32),
            pltpu.SemaphoreType.DMA,
            pltpu.SemaphoreType.DMA,
            pltpu.SemaphoreType.DMA,
            pltpu.SemaphoreType.DMA,
            pltpu.SemaphoreType.DMA,
        ],
    )
    def sc_kernel(idx_hbm, tbl2_hbm, out_hbm, idx0, idx1, idxp0, idxp1,
                  gbuf0, gbuf1, obuf0, obuf1, isem, gsem0, gsem1, osem0,
                  osem1):
        wid = lax.axis_index("s") * num_cores + lax.axis_index("c")
        b0 = wid * b_per_w
        j0 = b0 * n_seq
        idxb = (idx0, idx1)
        idxp = (idxp0, idxp1)
        gbuf = (gbuf0, gbuf1)
        obuf = (obuf0, obuf1)
        gsem = (gsem0, gsem1)
        osem = (osem0, osem1)

        def gather(k, bl, buf):
            src = idxb[k % 2]
            for h in range(2):
                base = bl * n_seq + h * (n_seq - _SG)
                for g in range(_SG // 16):
                    v = src[pl.ds(base + g * 16, 16)]
                    idxp[buf][h, pl.ds(g * 16, 16)] = v >> 1
            pltpu.async_copy(
                tbl2_hbm.at[idxp[buf].at[0]], gbuf[buf].at[pl.ds(0, _SG)],
                gsem[buf],
            )
            pltpu.async_copy(
                tbl2_hbm.at[idxp[buf].at[1]], gbuf[buf].at[pl.ds(_SG, _SG)],
                gsem[buf],
            )

        def wait_gather(buf):
            for h in range(2):
                pltpu.make_async_copy(
                    tbl2_hbm.at[idxp[buf].at[h]],
                    gbuf[buf].at[pl.ds(h * _SG, _SG)],
                    gsem[buf],
                ).wait()

        def sel_one(buf, r, grow, w_odd, w_even):
            for g in range(D_MODEL // 16):
                ve = gbuf[buf][grow, pl.ds(g * 16, 16)]
                vo = gbuf[buf][grow, pl.ds(D_MODEL + g * 16, 16)]
                obuf[buf][r, pl.ds(g * 16, 16)] = ve * w_even + vo * w_odd

        def sel(k, bl, buf):
            src = idxb[k % 2]

            @plsc.parallel_loop(0, (n_seq - 8) // 16, unroll=2)
            def sel_grp(j):
                vpar = jnp.bitwise_and(
                    src[pl.ds(bl * n_seq + j * 16, 16)], 1
                ).astype(jnp.float32) * _SCALE
                for l in range(16):
                    r = j * 16 + l
                    grow = jnp.where(r < _SG, r, r + shift)
                    w_odd = _lane_splat(vpar, l)
                    sel_one(buf, r, grow, w_odd, _SCALE - w_odd)
            vpar_t = jnp.bitwise_and(
                src[pl.ds(bl * n_seq + n_seq - 16, 16)], 1
            ).astype(jnp.float32) * _SCALE
            for l in range(8, 16):
                r = n_seq - 16 + l
                w_odd = _lane_splat(vpar_t, l)
                sel_one(buf, r, r + shift, w_odd, _SCALE - w_odd)

        pltpu.sync_copy(idx_hbm.at[pl.ds(j0, stage_len)], idx0)
        for k in range(n_blocks):
            if k + 1 < n_blocks:
                pltpu.async_copy(
                    idx_hbm.at[pl.ds(j0 + (k + 1) * stage_len, stage_len)],
                    idxb[(k + 1) % 2],
                    isem,
                )
            gather(k, 0, 0)

            def do_pair(p, carry):
                for buf in range(2):
                    bl = 2 * p + buf
                    wait_gather(buf)

                    @pl.when(bl + 1 < _BSTAGE)
                    def _():
                        gather(k, bl + 1, buf ^ 1)

                    if k > 0:
                        pltpu.make_async_copy(
                            obuf[buf], out_hbm.at[pl.ds(0, n_seq)], osem[buf]
                        ).wait()
                    else:
                        @pl.when(p >= 1)
                        def _():
                            pltpu.make_async_copy(
                                obuf[buf], out_hbm.at[pl.ds(0, n_seq)],
                                osem[buf],
                            ).wait()

                    sel(k, bl, buf)
                    b = b0 + k * _BSTAGE + bl
                    pltpu.async_copy(
                        obuf[buf], out_hbm.at[pl.ds(b * n_seq, n_seq)],
                        osem[buf],
                    )
                return carry

            lax.fori_loop(0, _BSTAGE // 2, do_pair, 0)
            if k + 1 < n_blocks:
                pltpu.make_async_copy(
                    idx_hbm.at[pl.ds(j0, stage_len)], idxb[(k + 1) % 2], isem
                ).wait()

        for buf in range(2):
            pltpu.make_async_copy(
                obuf[buf], out_hbm.at[pl.ds(0, n_seq)], osem[buf]
            ).wait()

    return sc_kernel


def kernel(x, table):
    n_batch, n_seq = x.shape
    flat_idx = x.reshape(n_batch * n_seq).astype(jnp.int32)
    tbl2 = table.reshape(table.shape[0] // 2, 2 * D_MODEL)
    out = _make_sc_gather(n_batch, n_seq)(flat_idx, tbl2)
    return out.reshape(n_batch, n_seq, D_MODEL)


# speedup vs baseline: 1.5680x; 1.0345x over previous
import functools
import math

import jax
import jax.numpy as jnp
from jax import lax
from jax.experimental import pallas as pl
from jax.experimental.pallas import tpu as pltpu
from jax.experimental.pallas import tpu_sc as plsc

D_MODEL = 64
_SCALE = math.sqrt(D_MODEL)
_BSTAGE = 32
_SG = 112

_SPLAT_DNUMS = lax.GatherDimensionNumbers(
    offset_dims=(), collapsed_slice_dims=(0,), start_index_map=(0,)
)


def _lane_splat(v, l):
    return lax.gather(
        v,
        jnp.full((16, 1), l, jnp.int32),
        _SPLAT_DNUMS,
        (1,),
        mode=lax.GatherScatterMode.PROMISE_IN_BOUNDS,
    )


@functools.lru_cache(maxsize=None)
def _make_sc_gather(n_batch: int, n_seq: int):
    info = plsc.get_sparse_core_info()
    num_cores, num_subcores = info.num_cores, info.num_subcores
    num_workers = num_cores * num_subcores
    assert n_batch % (num_workers * _BSTAGE) == 0
    b_per_w = n_batch // num_workers
    n_blocks = b_per_w // _BSTAGE
    stage_len = _BSTAGE * n_seq
    shift = 2 * _SG - n_seq
    mesh = plsc.VectorSubcoreMesh(core_axis_name="c", subcore_axis_name="s")

    @functools.partial(
        pl.kernel,
        mesh=mesh,
        out_type=jax.ShapeDtypeStruct((n_batch * n_seq, D_MODEL), jnp.float32),
        scratch_types=[
            pltpu.VMEM((stage_len,), jnp.int32),
            pltpu.VMEM((stage_len,), jnp.int32),
            pltpu.VMEM((n_seq,), jnp.int32),
            pltpu.VMEM((n_seq,), jnp.int32),
            pltpu.VMEM((n_seq, 2 * D_MODEL), jnp.float32),
            pltpu.VMEM((n_seq, 2 * D_MODEL), jnp.float32),
            pltpu.VMEM((n_seq, D_MODEL), jnp.float32),
            pltpu.VMEM((n_seq, D_MODEL), jnp.float32),
            pltpu.SemaphoreType.DMA,
            pltpu.SemaphoreType.DMA,
            pltpu.SemaphoreType.DMA,
            pltpu.SemaphoreType.DMA,
            pltpu.SemaphoreType.DMA,
        ],
    )
    def sc_kernel(idx_hbm, tbl2_hbm, out_hbm, idx0, idx1, idxp0, idxp1,
                  gbuf0, gbuf1, obuf0, obuf1, isem, gsem0, gsem1, osem0,
                  osem1):
        wid = lax.axis_index("s") * num_cores + lax.axis_index("c")
        b0 = wid * b_per_w
        j0 = b0 * n_seq
        idxb = (idx0, idx1)
        idxp = (idxp0, idxp1)
        gbuf = (gbuf0, gbuf1)
        obuf = (obuf0, obuf1)
        gsem = (gsem0, gsem1)
        osem = (osem0, osem1)

        def gather(k, bl, buf):
            src = idxb[k % 2]
            base = bl * n_seq
            for g in range(n_seq // 16):
                v = src[pl.ds(base + g * 16, 16)]
                idxp[buf][pl.ds(g * 16, 16)] = v >> 1
            v = src[pl.ds(base + n_seq - 16, 16)]
            idxp[buf][pl.ds(n_seq - 16, 16)] = v >> 1
            pltpu.async_copy(tbl2_hbm.at[idxp[buf]], gbuf[buf], gsem[buf])

        def wait_gather(buf):
            pltpu.make_async_copy(
                tbl2_hbm.at[idxp[buf]], gbuf[buf], gsem[buf]
            ).wait()

        def sel_one(buf, r, grow, w_odd, w_even):
            for g in range(D_MODEL // 16):
                ve = gbuf[buf][grow, pl.ds(g * 16, 16)]
                vo = gbuf[buf][grow, pl.ds(D_MODEL + g * 16, 16)]
                obuf[buf][r, pl.ds(g * 16, 16)] = ve * w_even + vo * w_odd

        def sel(k, bl, buf):
            src = idxb[k % 2]

            @plsc.parallel_loop(0, (n_seq - 8) // 16, unroll=2)
            def sel_grp(j):
                vpar = jnp.bitwise_and(
                    src[pl.ds(bl * n_seq + j * 16, 16)], 1
                ).astype(jnp.float32) * _SCALE
                for l in range(16):
                    r = j * 16 + l
                    w_odd = _lane_splat(vpar, l)
                    sel_one(buf, r, r, w_odd, _SCALE - w_odd)
            vpar_t = jnp.bitwise_and(
                src[pl.ds(bl * n_seq + n_seq - 16, 16)], 1
            ).astype(jnp.float32) * _SCALE
            for l in range(8, 16):
                r = n_seq - 16 + l
                w_odd = _lane_splat(vpar_t, l)
                sel_one(buf, r, r, w_odd, _SCALE - w_odd)

        pltpu.sync_copy(idx_hbm.at[pl.ds(j0, stage_len)], idx0)
        for k in range(n_blocks):
            if k + 1 < n_blocks:
                pltpu.async_copy(
                    idx_hbm.at[pl.ds(j0 + (k + 1) * stage_len, stage_len)],
                    idxb[(k + 1) % 2],
                    isem,
                )
            gather(k, 0, 0)

            def do_pair(p, carry):
                for buf in range(2):
                    bl = 2 * p + buf
                    wait_gather(buf)

                    @pl.when(bl + 1 < _BSTAGE)
                    def _():
                        gather(k, bl + 1, buf ^ 1)

                    if k > 0:
                        pltpu.make_async_copy(
                            obuf[buf], out_hbm.at[pl.ds(0, n_seq)], osem[buf]
                        ).wait()
                    else:
                        @pl.when(p >= 1)
                        def _():
                            pltpu.make_async_copy(
                                obuf[buf], out_hbm.at[pl.ds(0, n_seq)],
                                osem[buf],
                            ).wait()

                    sel(k, bl, buf)
                    b = b0 + k * _BSTAGE + bl
                    pltpu.async_copy(
                        obuf[buf], out_hbm.at[pl.ds(b * n_seq, n_seq)],
                        osem[buf],
                    )
                return carry

            lax.fori_loop(0, _BSTAGE // 2, do_pair, 0)
            if k + 1 < n_blocks:
                pltpu.make_async_copy(
                    idx_hbm.at[pl.ds(j0, stage_len)], idxb[(k + 1) % 2], isem
                ).wait()

        for buf in range(2):
            pltpu.make_async_copy(
                obuf[buf], out_hbm.at[pl.ds(0, n_seq)], osem[buf]
            ).wait()

    return sc_kernel


def kernel(x, table):
    n_batch, n_seq = x.shape
    flat_idx = x.reshape(n_batch * n_seq).astype(jnp.int32)
    tbl2 = table.reshape(table.shape[0] // 2, 2 * D_MODEL)
    out = _make_sc_gather(n_batch, n_seq)(flat_idx, tbl2)
    return out.reshape(n_batch, n_seq, D_MODEL)


# speedup vs baseline: 1.5703x; 1.0014x over previous
import functools
import math

import jax
import jax.numpy as jnp
from jax import lax
from jax.experimental import pallas as pl
from jax.experimental.pallas import tpu as pltpu
from jax.experimental.pallas import tpu_sc as plsc

D_MODEL = 64
_SCALE = math.sqrt(D_MODEL)
_BSTAGE = 32

_SPLAT_DNUMS = lax.GatherDimensionNumbers(
    offset_dims=(), collapsed_slice_dims=(0,), start_index_map=(0,)
)


def _lane_splat(v, l):
    return lax.gather(
        v,
        jnp.full((16, 1), l, jnp.int32),
        _SPLAT_DNUMS,
        (1,),
        mode=lax.GatherScatterMode.PROMISE_IN_BOUNDS,
    )


@functools.lru_cache(maxsize=None)
def _make_sc_gather(n_batch: int, n_seq: int):
    info = plsc.get_sparse_core_info()
    num_cores, num_subcores = info.num_cores, info.num_subcores
    num_workers = num_cores * num_subcores
    assert n_batch % (num_workers * _BSTAGE) == 0
    b_per_w = n_batch // num_workers
    n_blocks = b_per_w // _BSTAGE
    stage_len = _BSTAGE * n_seq
    mesh = plsc.VectorSubcoreMesh(core_axis_name="c", subcore_axis_name="s")

    @functools.partial(
        pl.kernel,
        mesh=mesh,
        out_type=jax.ShapeDtypeStruct((n_batch * n_seq, D_MODEL), jnp.float32),
        scratch_types=[
            pltpu.VMEM((stage_len,), jnp.int32),
            pltpu.VMEM((stage_len,), jnp.int32),
            pltpu.VMEM((n_seq,), jnp.int32),
            pltpu.VMEM((n_seq,), jnp.int32),
            pltpu.VMEM((n_seq, 2 * D_MODEL), jnp.float32),
            pltpu.VMEM((n_seq, 2 * D_MODEL), jnp.float32),
            pltpu.VMEM((n_seq, D_MODEL), jnp.float32),
            pltpu.VMEM((n_seq, D_MODEL), jnp.float32),
            pltpu.SemaphoreType.DMA,
            pltpu.SemaphoreType.DMA,
            pltpu.SemaphoreType.DMA,
            pltpu.SemaphoreType.DMA,
            pltpu.SemaphoreType.DMA,
        ],
    )
    def sc_kernel(idx_hbm, tbl2_hbm, out_hbm, idx0, idx1, idxp0, idxp1,
                  gbuf0, gbuf1, obuf0, obuf1, isem, gsem0, gsem1, osem0,
                  osem1):
        wid = lax.axis_index("s") * num_cores + lax.axis_index("c")
        b0 = wid * b_per_w
        j0 = b0 * n_seq
        idxb = (idx0, idx1)
        idxp = (idxp0, idxp1)
        gbuf = (gbuf0, gbuf1)
        obuf = (obuf0, obuf1)
        gsem = (gsem0, gsem1)
        osem = (osem0, osem1)

        def gather(k, bl, buf):
            src = idxb[k % 2]
            base = bl * n_seq
            for g in range(n_seq // 16):
                v = src[pl.ds(base + g * 16, 16)]
                idxp[buf][pl.ds(g * 16, 16)] = v >> 1
            v = src[pl.ds(base + n_seq - 16, 16)]
            idxp[buf][pl.ds(n_seq - 16, 16)] = v >> 1
            pltpu.async_copy(tbl2_hbm.at[idxp[buf]], gbuf[buf], gsem[buf])

        def wait_gather(buf):
            pltpu.make_async_copy(
                tbl2_hbm.at[idxp[buf]], gbuf[buf], gsem[buf]
            ).wait()

        def sel_one(buf, r, grow, w_odd, w_even):
            for g in range(D_MODEL // 16):
                ve = gbuf[buf][grow, pl.ds(g * 16, 16)]
                vo = gbuf[buf][grow, pl.ds(D_MODEL + g * 16, 16)]
                obuf[buf][r, pl.ds(g * 16, 16)] = ve * w_even + vo * w_odd

        def sel(k, bl, buf):
            src = idxb[k % 2]

            @plsc.parallel_loop(0, (n_seq - 8) // 16, unroll=2)
            def sel_grp(j):
                vpar = jnp.bitwise_and(
                    src[pl.ds(bl * n_seq + j * 16, 16)], 1
                ).astype(jnp.float32) * _SCALE
                for l in range(16):
                    r = j * 16 + l
                    w_odd = _lane_splat(vpar, l)
                    sel_one(buf, r, r, w_odd, _SCALE - w_odd)
            vpar_t = jnp.bitwise_and(
                src[pl.ds(bl * n_seq + n_seq - 16, 16)], 1
            ).astype(jnp.float32) * _SCALE
            for l in range(8, 16):
                r = n_seq - 16 + l
                w_odd = _lane_splat(vpar_t, l)
                sel_one(buf, r, r, w_odd, _SCALE - w_odd)

        pltpu.sync_copy(idx_hbm.at[pl.ds(j0, stage_len)], idx0)
        for k in range(n_blocks):
            if k + 1 < n_blocks:
                pltpu.async_copy(
                    idx_hbm.at[pl.ds(j0 + (k + 1) * stage_len, stage_len)],
                    idxb[(k + 1) % 2],
                    isem,
                )
            gather(k, 0, 0)

            def do_pair(p, carry):
                for buf in range(2):
                    bl = 2 * p + buf
                    wait_gather(buf)

                    @pl.when(bl + 1 < _BSTAGE)
                    def _():
                        gather(k, bl + 1, buf ^ 1)

                    if k > 0:
                        pltpu.make_async_copy(
                            obuf[buf], out_hbm.at[pl.ds(0, n_seq)], osem[buf]
                        ).wait()
                    else:
                        @pl.when(p >= 1)
                        def _():
                            pltpu.make_async_copy(
                                obuf[buf], out_hbm.at[pl.ds(0, n_seq)],
                                osem[buf],
                            ).wait()

                    sel(k, bl, buf)
                    b = b0 + k * _BSTAGE + bl
                    pltpu.async_copy(
                        obuf[buf], out_hbm.at[pl.ds(b * n_seq, n_seq)],
                        osem[buf],
                    )
                return carry

            lax.fori_loop(0, _BSTAGE // 2, do_pair, 0)
            if k + 1 < n_blocks:
                pltpu.make_async_copy(
                    idx_hbm.at[pl.ds(j0, stage_len)], idxb[(k + 1) % 2], isem
                ).wait()

        for buf in range(2):
            pltpu.make_async_copy(
                obuf[buf], out_hbm.at[pl.ds(0, n_seq)], osem[buf]
            ).wait()

    return sc_kernel


def kernel(x, table):
    n_batch, n_seq = x.shape
    flat_idx = x.reshape(n_batch * n_seq).astype(jnp.int32)
    tbl2 = table.reshape(table.shape[0] // 2, 2 * D_MODEL)
    out = _make_sc_gather(n_batch, n_seq)(flat_idx, tbl2)
    return out.reshape(n_batch, n_seq, D_MODEL)


# speedup vs baseline: 1.5887x; 1.0117x over previous
import functools
import math

import jax
import jax.numpy as jnp
from jax import lax
from jax.experimental import pallas as pl
from jax.experimental.pallas import tpu as pltpu
from jax.experimental.pallas import tpu_sc as plsc

D_MODEL = 64
_SCALE = math.sqrt(D_MODEL)
_BSTAGE = 32

_SPLAT_DNUMS = lax.GatherDimensionNumbers(
    offset_dims=(), collapsed_slice_dims=(0,), start_index_map=(0,)
)


def _lane_splat(v, l):
    return lax.gather(
        v,
        jnp.full((16, 1), l, jnp.int32),
        _SPLAT_DNUMS,
        (1,),
        mode=lax.GatherScatterMode.PROMISE_IN_BOUNDS,
    )


@functools.lru_cache(maxsize=None)
def _make_sc_gather(n_batch: int, n_seq: int):
    info = plsc.get_sparse_core_info()
    num_cores, num_subcores = info.num_cores, info.num_subcores
    num_workers = num_cores * num_subcores
    assert n_batch % (num_workers * _BSTAGE) == 0
    b_per_w = n_batch // num_workers
    n_blocks = b_per_w // _BSTAGE
    stage_len = _BSTAGE * n_seq
    mesh = plsc.VectorSubcoreMesh(core_axis_name="c", subcore_axis_name="s")

    @functools.partial(
        pl.kernel,
        mesh=mesh,
        out_type=jax.ShapeDtypeStruct((n_batch * n_seq, D_MODEL), jnp.float32),
        scratch_types=[
            pltpu.VMEM((stage_len,), jnp.int32),
            pltpu.VMEM((stage_len,), jnp.int32),
            pltpu.VMEM((n_seq,), jnp.int32),
            pltpu.VMEM((n_seq,), jnp.int32),
            pltpu.VMEM((n_seq, 2 * D_MODEL), jnp.float32),
            pltpu.VMEM((n_seq, 2 * D_MODEL), jnp.float32),
            pltpu.VMEM((n_seq, D_MODEL), jnp.float32),
            pltpu.VMEM((n_seq, D_MODEL), jnp.float32),
            pltpu.SemaphoreType.DMA,
            pltpu.SemaphoreType.DMA,
            pltpu.SemaphoreType.DMA,
            pltpu.SemaphoreType.DMA,
            pltpu.SemaphoreType.DMA,
        ],
    )
    def sc_kernel(idx_hbm, tbl2_hbm, out_hbm, idx0, idx1, idxp0, idxp1,
                  gbuf0, gbuf1, obuf0, obuf1, isem, gsem0, gsem1, osem0,
                  osem1):
        wid = lax.axis_index("s") * num_cores + lax.axis_index("c")
        b0 = wid * b_per_w
        j0 = b0 * n_seq
        idxb = (idx0, idx1)
        idxp = (idxp0, idxp1)
        gbuf = (gbuf0, gbuf1)
        obuf = (obuf0, obuf1)
        gsem = (gsem0, gsem1)
        osem = (osem0, osem1)

        def gather(k, bl, buf):
            src = idxb[k % 2]
            base = bl * n_seq
            for g in range(n_seq // 16):
                v = src[pl.ds(base + g * 16, 16)]
                idxp[buf][pl.ds(g * 16, 16)] = v >> 1
            v = src[pl.ds(base + n_seq - 16, 16)]
            idxp[buf][pl.ds(n_seq - 16, 16)] = v >> 1
            pltpu.async_copy(tbl2_hbm.at[idxp[buf]], gbuf[buf], gsem[buf])

        def wait_gather(buf):
            pltpu.make_async_copy(
                tbl2_hbm.at[idxp[buf]], gbuf[buf], gsem[buf]
            ).wait()

        def sel_one(buf, r, grow, w_odd, w_even):
            for g in range(D_MODEL // 16):
                ve = gbuf[buf][grow, pl.ds(g * 16, 16)]
                vo = gbuf[buf][grow, pl.ds(D_MODEL + g * 16, 16)]
                obuf[buf][r, pl.ds(g * 16, 16)] = ve * w_even + vo * w_odd

        def sel(k, bl, buf):
            src = idxb[k % 2]

            @plsc.parallel_loop(0, (n_seq - 8) // 16, unroll=2)
            def sel_grp(j):
                vpar = jnp.bitwise_and(
                    src[pl.ds(bl * n_seq + j * 16, 16)], 1
                ).astype(jnp.float32) * _SCALE
                for l in range(16):
                    r = j * 16 + l
                    w_odd = _lane_splat(vpar, l)
                    sel_one(buf, r, r, w_odd, _SCALE - w_odd)
            vpar_t = jnp.bitwise_and(
                src[pl.ds(bl * n_seq + n_seq - 16, 16)], 1
            ).astype(jnp.float32) * _SCALE
            for l in range(8, 16):
                r = n_seq - 16 + l
                w_odd = _lane_splat(vpar_t, l)
                sel_one(buf, r, r, w_odd, _SCALE - w_odd)

        pltpu.sync_copy(idx_hbm.at[pl.ds(j0, stage_len)], idx0)
        for k in range(n_blocks):
            if k + 1 < n_blocks:
                pltpu.async_copy(
                    idx_hbm.at[pl.ds(j0 + (k + 1) * stage_len, stage_len)],
                    idxb[(k + 1) % 2],
                    isem,
                )
            gather(k, 0, 0)

            def do_pair(p, carry):
                for buf in range(2):
                    bl = 2 * p + buf

                    @pl.when(bl + 1 < _BSTAGE)
                    def _():
                        gather(k, bl + 1, buf ^ 1)

                    wait_gather(buf)

                    if k > 0:
                        pltpu.make_async_copy(
                            obuf[buf], out_hbm.at[pl.ds(0, n_seq)], osem[buf]
                        ).wait()
                    else:
                        @pl.when(p >= 1)
                        def _():
                            pltpu.make_async_copy(
                                obuf[buf], out_hbm.at[pl.ds(0, n_seq)],
                                osem[buf],
                            ).wait()

                    sel(k, bl, buf)
                    b = b0 + k * _BSTAGE + bl
                    pltpu.async_copy(
                        obuf[buf], out_hbm.at[pl.ds(b * n_seq, n_seq)],
                        osem[buf],
                    )
                return carry

            lax.fori_loop(0, _BSTAGE // 2, do_pair, 0)
            if k + 1 < n_blocks:
                pltpu.make_async_copy(
                    idx_hbm.at[pl.ds(j0, stage_len)], idxb[(k + 1) % 2], isem
                ).wait()

        for buf in range(2):
            pltpu.make_async_copy(
                obuf[buf], out_hbm.at[pl.ds(0, n_seq)], osem[buf]
            ).wait()

    return sc_kernel


def kernel(x, table):
    n_batch, n_seq = x.shape
    flat_idx = x.reshape(n_batch * n_seq).astype(jnp.int32)
    tbl2 = table.reshape(table.shape[0] // 2, 2 * D_MODEL)
    out = _make_sc_gather(n_batch, n_seq)(flat_idx, tbl2)
    return out.reshape(n_batch, n_seq, D_MODEL)
